# Initial kernel scaffold; baseline (speedup 1.0000x reference)
import jax
import jax.numpy as jnp
from jax.experimental import pallas as pl


def kernel(x, edge_index_a, edge_index_b):
    raise NotImplementedError("write your pallas kernel here")


# speedup vs baseline: 4.0040x; 4.0040x over previous
import functools

import jax
import jax.numpy as jnp
from jax import lax
from jax.experimental import pallas as pl
from jax.experimental.pallas import tpu as pltpu
from jax.experimental.pallas import tpu_sc as plsc

NS = 16
L = 16
BLK = 128


def _sc_kernel_body(n_nodes, half, bpt, xflat, colb, rowb, cola, rowa, zeros,
                    h1, out, acc, colv, rowv, rows_v, gsem):
  c = lax.axis_index("c")
  s = lax.axis_index("s")
  n_acc = acc.shape[0]
  stripe = n_acc // NS

  def do_pass(col_hbm, row_hbm, src_hbm, dst_hbm):
    pltpu.sync_copy(zeros.at[pl.ds(0, stripe)],
                    acc.at[pl.ds(s * stripe, stripe)])
    plsc.subcore_barrier()

    base = s * bpt

    def chunk(j, carry):
      pltpu.sync_copy(col_hbm.at[base + j, 0], colv)
      pltpu.sync_copy(row_hbm.at[base + j, 0], rowv)
      off = c * n_acc
      for i in range(BLK // L):
        colv[pl.ds(i * L, L)] = colv[pl.ds(i * L, L)] + off
      pltpu.async_copy(src_hbm.at[colv], rows_v, gsem).wait()
      pltpu.sync_copy(rows_v, acc.at[rowv], add=True)
      return carry

    lax.fori_loop(0, bpt, chunk, 0, unroll=False)
    plsc.subcore_barrier()
    pltpu.sync_copy(acc.at[pl.ds(s * stripe, stripe)],
                    dst_hbm.at[pl.ds(c * n_acc + s * stripe, stripe)])
    plsc.subcore_barrier()

  do_pass(colb, rowb, xflat, h1)
  do_pass(cola, rowa, h1, out)


def kernel(x, edge_index_a, edge_index_b):
  n, d = x.shape
  half = d // 2
  e = edge_index_a.shape[1]

  total_blocks = -(-e // BLK)
  bpt = -(-total_blocks // NS)
  e_pad = bpt * NS * BLK
  pad = e_pad - e

  n_acc = -(-n // (8 * NS)) * 8 * NS

  padrows = jnp.zeros((n_acc - n, half), jnp.float32)
  xflat = jnp.concatenate(
      [x[:, :half], padrows, x[:, half:], padrows], axis=0)

  def prep(idx, fill):
    p = jnp.concatenate([idx, jnp.full((pad,), fill, jnp.int32)])
    return p.reshape(NS * bpt, 1, BLK)

  colb = prep(edge_index_b[1], 0)
  rowb = prep(edge_index_b[0], n)
  cola = prep(edge_index_a[1], 0)
  rowa = prep(edge_index_a[0], n)

  zeros = jnp.zeros((n_acc // NS, half), jnp.float32)

  mesh = plsc.VectorSubcoreMesh(core_axis_name="c", subcore_axis_name="s")
  fn = pl.kernel(
      functools.partial(_sc_kernel_body, n, half, bpt),
      out_type=(
          jax.ShapeDtypeStruct((2 * n_acc, half), jnp.float32),
          jax.ShapeDtypeStruct((2 * n_acc, half), jnp.float32),
      ),
      mesh=mesh,
      scratch_types=[
          pltpu.VMEM_SHARED((n_acc, half), jnp.float32),
          pltpu.VMEM((BLK,), jnp.int32),
          pltpu.VMEM((BLK,), jnp.int32),
          pltpu.VMEM((BLK, half), jnp.float32),
          pltpu.SemaphoreType.DMA,
      ],
      compiler_params=pltpu.CompilerParams(use_tc_tiling_on_sc=False),
  )
  _, outflat = fn(xflat, colb, rowb, cola, rowa, zeros)
  return jnp.concatenate([outflat[:n], outflat[n_acc:n_acc + n]], axis=1)


# speedup vs baseline: 5.0345x; 1.2574x over previous
import functools

import jax
import jax.numpy as jnp
from jax import lax
from jax.experimental import pallas as pl
from jax.experimental.pallas import tpu as pltpu
from jax.experimental.pallas import tpu_sc as plsc

NS = 16
BLK = 128
G = 4


def _sc_body(n_acc, ng, xflat, colb, rowb, cola, rowa, zeros, dummy,
             h1, out, acc, cv0, wv0, cv1, wv1, rv0, rv1,
             gsem0, gsem1, ssem0, ssem1):
  c = lax.axis_index("c")
  s = lax.axis_index("s")
  stripe = n_acc // NS
  t_pairs = ng // 2

  def wait(sem):
    pltpu.make_async_copy(dummy, rv0, sem).wait()

  def do_pass(col_hbm, row_hbm, src_hbm, dst_hbm):
    pltpu.sync_copy(zeros, acc.at[pl.ds(s * stripe, stripe)])
    plsc.subcore_barrier()
    base = s * ng

    def load(gidx, cv, wv):
      pltpu.sync_copy(col_hbm.at[c, gidx], cv)
      pltpu.sync_copy(row_hbm.at[gidx], wv)

    def gather(cv, rv, sem):
      for g in range(G):
        pltpu.async_copy(src_hbm.at[cv.at[g]], rv.at[g], sem)

    def scatter(rv, wv, sem):
      for g in range(G):
        pltpu.async_copy(rv.at[g], acc.at[wv.at[g]], sem, add=True)

    load(base, cv0, wv0)
    gather(cv0, rv0, gsem0)
    load(base + 1, cv1, wv1)
    gather(cv1, rv1, gsem1)
    wait(gsem0)
    scatter(rv0, wv0, ssem0)
    wait(ssem0)
    load(base + 2, cv0, wv0)
    gather(cv0, rv0, gsem0)
    wait(gsem1)
    scatter(rv1, wv1, ssem1)

    def body(t, carry):
      g = base + 2 * t
      wait(ssem1)
      load(g + 1, cv1, wv1)
      gather(cv1, rv1, gsem1)
      wait(gsem0)
      scatter(rv0, wv0, ssem0)
      wait(ssem0)

      @pl.when(t < t_pairs - 1)
      def _():
        load(g + 2, cv0, wv0)
        gather(cv0, rv0, gsem0)

      wait(gsem1)
      scatter(rv1, wv1, ssem1)
      return carry

    lax.fori_loop(1, t_pairs, body, 0, unroll=False)
    wait(ssem1)
    plsc.subcore_barrier()
    pltpu.sync_copy(acc.at[pl.ds(s * stripe, stripe)],
                    dst_hbm.at[pl.ds(c * n_acc + s * stripe, stripe)])
    plsc.subcore_barrier()

  do_pass(colb, rowb, xflat, h1)
  do_pass(cola, rowa, h1, out)


def kernel(x, edge_index_a, edge_index_b):
  n, d = x.shape
  half = d // 2
  e = edge_index_a.shape[1]

  blocks = -(-e // BLK)
  ng = -(-blocks // (NS * G))
  ng = ng + (ng % 2)
  e_pad = NS * ng * G * BLK
  pad = e_pad - e

  n_acc = -(-n // (8 * NS)) * 8 * NS

  padrows = jnp.zeros((n_acc - n, half), jnp.float32)
  xflat = jnp.concatenate(
      [x[:, :half], padrows, x[:, half:], padrows], axis=0)

  def prep_cols(idx):
    p = jnp.concatenate([idx, jnp.zeros((pad,), jnp.int32)])
    both = jnp.stack([p, p + n_acc])
    return both.reshape(2, NS * ng, G, BLK)

  def prep_rows(idx):
    p = jnp.concatenate([idx, jnp.full((pad,), n, jnp.int32)])
    return p.reshape(NS * ng, G, BLK)

  colb = prep_cols(edge_index_b[1])
  rowb = prep_rows(edge_index_b[0])
  cola = prep_cols(edge_index_a[1])
  rowa = prep_rows(edge_index_a[0])

  zeros = jnp.zeros((n_acc // NS, half), jnp.float32)
  dummy = jnp.zeros((G, BLK, half), jnp.float32)

  mesh = plsc.VectorSubcoreMesh(core_axis_name="c", subcore_axis_name="s")
  fn = pl.kernel(
      functools.partial(_sc_body, n_acc, ng),
      out_type=(
          jax.ShapeDtypeStruct((2 * n_acc, half), jnp.float32),
          jax.ShapeDtypeStruct((2 * n_acc, half), jnp.float32),
      ),
      mesh=mesh,
      scratch_types=[
          pltpu.VMEM_SHARED((n_acc, half), jnp.float32),
          pltpu.VMEM((G, BLK), jnp.int32),
          pltpu.VMEM((G, BLK), jnp.int32),
          pltpu.VMEM((G, BLK), jnp.int32),
          pltpu.VMEM((G, BLK), jnp.int32),
          pltpu.VMEM((G, BLK, half), jnp.float32),
          pltpu.VMEM((G, BLK, half), jnp.float32),
          pltpu.SemaphoreType.DMA,
          pltpu.SemaphoreType.DMA,
          pltpu.SemaphoreType.DMA,
          pltpu.SemaphoreType.DMA,
      ],
      compiler_params=pltpu.CompilerParams(use_tc_tiling_on_sc=False),
  )
  _, outflat = fn(xflat, colb, rowb, cola, rowa, zeros, dummy)
  return jnp.concatenate([outflat[:n], outflat[n_acc:n_acc + n]], axis=1)
